# baseline (speedup 1.0000x reference)
import functools

import jax
import jax.numpy as jnp
from jax import lax
from jax.experimental import pallas as pl
from jax.experimental.pallas import tpu as pltpu
from jax.experimental.pallas import tpu_sc as plsc

BATCH = 16384
NUM_RATINGS = 5
LANES = 16
NUM_CORES = 1
NUM_SUBCORES = 8
NUM_WORKERS = NUM_CORES * NUM_SUBCORES
B_PER_W = BATCH // NUM_WORKERS
VECS_PER_W = B_PER_W // LANES
N_CHUNKS = 4


def _make_sc_kernel():
    mesh = plsc.VectorSubcoreMesh(
        core_axis_name="c", subcore_axis_name="s", num_cores=NUM_CORES, num_subcores=NUM_SUBCORES
    )

    @functools.partial(
        pl.kernel,
        mesh=mesh,
        out_type=jax.ShapeDtypeStruct((BATCH,), jnp.float32),
        compiler_params=pltpu.CompilerParams(needs_layout_passes=False),
        scratch_types=[
            pltpu.VMEM((LANES,), jnp.float32),
            pltpu.VMEM((B_PER_W,), jnp.int32),
            pltpu.VMEM((B_PER_W,), jnp.float32),
            pltpu.SemaphoreType.DMA,
            pltpu.SemaphoreType.DMA,
            pltpu.SemaphoreType.DMA,
        ],
    )
    def sc_gather(table_hbm, idx_hbm, out_hbm, tab_v, idx_v, out_v,
                  sem_tab, sem_idx, sem_out):
        wid = lax.axis_index("s") * NUM_CORES + lax.axis_index("c")
        base = wid * B_PER_W
        tab_copy = pltpu.async_copy(
            table_hbm, tab_v.at[pl.ds(0, NUM_RATINGS)], sem_tab)
        idx_copy = pltpu.async_copy(
            idx_hbm.at[pl.ds(base, B_PER_W)], idx_v, sem_idx)
        tab_copy.wait()
        idx_copy.wait()
        out_copies = []
        chunk_vecs = VECS_PER_W // N_CHUNKS
        chunk_elems = chunk_vecs * LANES
        for c in range(N_CHUNKS):
            for i in range(c * chunk_vecs, (c + 1) * chunk_vecs):
                idx = idx_v[pl.ds(i * LANES, LANES)]
                out_v[pl.ds(i * LANES, LANES)] = plsc.load_gather(
                    tab_v, [idx])
            out_copies.append(pltpu.async_copy(
                out_v.at[pl.ds(c * chunk_elems, chunk_elems)],
                out_hbm.at[pl.ds(base + c * chunk_elems, chunk_elems)],
                sem_out))
        for cp in out_copies:
            cp.wait()

    return sc_gather


_sc_gather = _make_sc_kernel()


def kernel(batch, y_dict):
    return _sc_gather(y_dict, batch.astype(jnp.int32))


# speedup vs baseline: 1.0436x; 1.0436x over previous
import functools

import jax
import jax.numpy as jnp
from jax import lax
from jax.experimental import pallas as pl
from jax.experimental.pallas import tpu as pltpu
from jax.experimental.pallas import tpu_sc as plsc

BATCH = 16384
NUM_RATINGS = 5
LANES = 16
NUM_CORES = 1
NUM_SUBCORES = 16
NUM_WORKERS = NUM_CORES * NUM_SUBCORES
B_PER_W = BATCH // NUM_WORKERS
VECS_PER_W = B_PER_W // LANES
N_CHUNKS = 4


def _make_sc_kernel():
    mesh = plsc.VectorSubcoreMesh(
        core_axis_name="c", subcore_axis_name="s", num_cores=NUM_CORES, num_subcores=NUM_SUBCORES
    )

    @functools.partial(
        pl.kernel,
        mesh=mesh,
        out_type=jax.ShapeDtypeStruct((BATCH,), jnp.float32),
        compiler_params=pltpu.CompilerParams(needs_layout_passes=False),
        scratch_types=[
            pltpu.VMEM((LANES,), jnp.float32),
            pltpu.VMEM((B_PER_W,), jnp.int32),
            pltpu.VMEM((B_PER_W,), jnp.float32),
            pltpu.SemaphoreType.DMA,
            pltpu.SemaphoreType.DMA,
            pltpu.SemaphoreType.DMA,
        ],
    )
    def sc_gather(table_hbm, idx_hbm, out_hbm, tab_v, idx_v, out_v,
                  sem_tab, sem_idx, sem_out):
        wid = lax.axis_index("s") * NUM_CORES + lax.axis_index("c")
        base = wid * B_PER_W
        tab_copy = pltpu.async_copy(
            table_hbm, tab_v.at[pl.ds(0, NUM_RATINGS)], sem_tab)
        idx_copy = pltpu.async_copy(
            idx_hbm.at[pl.ds(base, B_PER_W)], idx_v, sem_idx)
        tab_copy.wait()
        idx_copy.wait()
        out_copies = []
        chunk_vecs = VECS_PER_W // N_CHUNKS
        chunk_elems = chunk_vecs * LANES
        for c in range(N_CHUNKS):
            for i in range(c * chunk_vecs, (c + 1) * chunk_vecs):
                idx = idx_v[pl.ds(i * LANES, LANES)]
                out_v[pl.ds(i * LANES, LANES)] = plsc.load_gather(
                    tab_v, [idx])
            out_copies.append(pltpu.async_copy(
                out_v.at[pl.ds(c * chunk_elems, chunk_elems)],
                out_hbm.at[pl.ds(base + c * chunk_elems, chunk_elems)],
                sem_out))
        for cp in out_copies:
            cp.wait()

    return sc_gather


_sc_gather = _make_sc_kernel()


def kernel(batch, y_dict):
    return _sc_gather(y_dict, batch.astype(jnp.int32))


# speedup vs baseline: 12.7851x; 12.2505x over previous
import jax
import jax.numpy as jnp
from jax.experimental import pallas as pl
from jax.experimental.pallas import tpu as pltpu

BATCH = 16384
NUM_RATINGS = 5


def _lookup_body(tab_smem, batch_ref, out_ref):
    b = batch_ref[...]
    acc = jnp.full((BATCH,), tab_smem[0], jnp.float32)
    for k in range(1, NUM_RATINGS):
        acc = jnp.where(b == k, tab_smem[k], acc)
    out_ref[...] = acc


def kernel(batch, y_dict):
    return pl.pallas_call(
        _lookup_body,
        out_shape=jax.ShapeDtypeStruct((BATCH,), jnp.float32),
        in_specs=[
            pl.BlockSpec(memory_space=pltpu.SMEM),
            pl.BlockSpec((BATCH,), lambda: (0,)),
        ],
        out_specs=pl.BlockSpec((BATCH,), lambda: (0,)),
    )(y_dict, batch.astype(jnp.int32))
